# baseline (speedup 1.0000x reference)
import functools

import jax
import jax.numpy as jnp
from jax import lax
from jax.experimental import pallas as pl
from jax.experimental.pallas import tpu as pltpu
from jax.experimental.pallas import tpu_sc as plsc

L = 200
D = 64
NB = 4096
LANES = 16
NC = 2
NS = 16
NW = NC * NS
CHUNK = 256
NCHUNK = NB // CHUNK
MAXL_W = (L + NW - 1) // NW


@functools.cache
def _build():
    mesh = plsc.VectorSubcoreMesh(core_axis_name="c", subcore_axis_name="s")

    @functools.partial(
        pl.kernel,
        mesh=mesh,
        out_type=jax.ShapeDtypeStruct((L, D, NB), jnp.float32),
        scratch_types=[
            pltpu.VMEM((MAXL_W, NB // 128, 128), jnp.int32),
            pltpu.VMEM((MAXL_W, D * LANES // 128, 128), jnp.float32),
            pltpu.VMEM((CHUNK, D), jnp.float32),
            pltpu.VMEM((CHUNK, D), jnp.float32),
            pltpu.VMEM((D, CHUNK), jnp.float32),
            pltpu.VMEM((D, CHUNK), jnp.float32),
            pltpu.SemaphoreType.DMA,
            pltpu.SemaphoreType.DMA,
            pltpu.SemaphoreType.DMA,
            pltpu.SemaphoreType.DMA,
        ],
        compiler_params=pltpu.CompilerParams(
            use_tc_tiling_on_sc=False, needs_layout_passes=False
        ),
    )
    def run(seq_hbm, tok_hbm, pos_hbm, out_hbm,
            idx_all, pos_all, rows0, rows1, x0, x1,
            gsem0, gsem1, osem0, osem1):
        rows_v = (rows0, rows1)
        xout_v = (x0, x1)
        gsem = (gsem0, gsem1)
        osem = (osem0, osem1)
        w = lax.axis_index("s") * NC + lax.axis_index("c")
        n_l = (L - 1 - w) // NW + 1
        nj = n_l * NCHUNK

        def stage(i, _):
            li = w + i * NW
            pltpu.sync_copy(seq_hbm.at[li], idx_all.at[i])
            pltpu.sync_copy(pos_hbm.at[li], pos_all.at[i])
            return 0

        lax.fori_loop(0, n_l, stage, 0)

        iota = lax.iota(jnp.int32, LANES)
        row16 = [iota + b0 for b0 in range(0, CHUNK, LANES)]

        def fire_gathers(j, b):
            li = j // NCHUNK
            c = j % NCHUNK
            for k in range(CHUNK // 128):
                pltpu.async_copy(
                    tok_hbm.at[idx_all.at[li, c * 2 + k]],
                    rows_v[b].at[pl.ds(k * 128, 128)],
                    gsem[b],
                )

        def drain(dummy_src, dst, sem):
            pltpu.make_async_copy(dummy_src, dst, sem).wait()

        fire_gathers(jnp.int32(0), 0)

        def half_body(jj, _):
            for b in (0, 1):
                j = jj * 2 + b
                li = j // NCHUNK
                c = j % NCHUNK
                l = w + li * NW

                @pl.when(j + 1 < nj)
                def _():
                    fire_gathers(j + 1, 1 - b)

                drain(tok_hbm.at[pl.ds(0, CHUNK)], rows_v[b], gsem[b])

                @pl.when(j >= 2)
                def _():
                    drain(out_hbm.at[0, :, pl.ds(0, CHUNK)], xout_v[b],
                          osem[b])

                def d_body(d, _):
                    pv = pos_all[li, d >> 3, pl.ds((d & 7) * LANES, LANES)]
                    colv = jnp.full((LANES,), d, jnp.int32)
                    for s in range(CHUNK // LANES):
                        g = plsc.load_gather(rows_v[b], [row16[s], colv])
                        xout_v[b][d, pl.ds(s * LANES, LANES)] = g + pv
                    return 0

                lax.fori_loop(0, D, d_body, 0)

                pltpu.async_copy(
                    xout_v[b],
                    out_hbm.at[l, :, pl.ds(c * CHUNK, CHUNK)],
                    osem[b],
                )
            return 0

        lax.fori_loop(0, nj // 2, half_body, 0)

        for b in (0, 1):
            drain(out_hbm.at[0, :, pl.ds(0, CHUNK)], xout_v[b], osem[b])

    return run


def kernel(seq, token_table, pos_table):
    b, l = seq.shape
    d = token_table.shape[1]
    seq3 = seq.T.reshape(l, b // 128, 128)
    pos16 = jnp.broadcast_to(
        pos_table[:, :, None], (l, d, LANES)
    ).reshape(l, d * LANES // 128, 128)
    out = _build()(seq3, token_table, pos16)
    return jnp.transpose(out, (2, 0, 1))


# speedup vs baseline: 1.1957x; 1.1957x over previous
import functools

import jax
import jax.numpy as jnp
from jax import lax
from jax.experimental import pallas as pl
from jax.experimental.pallas import tpu as pltpu
from jax.experimental.pallas import tpu_sc as plsc

L = 200
D = 64
NB = 4096
LANES = 16
NC = 2
NS = 16
NW = NC * NS
CHUNK = 256
NCHUNK = NB // CHUNK
MAXL_W = (L + NW - 1) // NW


@functools.cache
def _build():
    mesh = plsc.VectorSubcoreMesh(core_axis_name="c", subcore_axis_name="s")

    @functools.partial(
        pl.kernel,
        mesh=mesh,
        out_type=jax.ShapeDtypeStruct((L, D, NB), jnp.float32),
        scratch_types=[
            pltpu.VMEM((MAXL_W, NB // 128, 128), jnp.int32),
            pltpu.VMEM((MAXL_W, D * LANES // 128, 128), jnp.float32),
            pltpu.VMEM((CHUNK, D), jnp.float32),
            pltpu.VMEM((CHUNK, D), jnp.float32),
            pltpu.VMEM((D, CHUNK), jnp.float32),
            pltpu.VMEM((D, CHUNK), jnp.float32),
            pltpu.SemaphoreType.DMA,
            pltpu.SemaphoreType.DMA,
            pltpu.SemaphoreType.DMA,
            pltpu.SemaphoreType.DMA,
        ],
        compiler_params=pltpu.CompilerParams(
            use_tc_tiling_on_sc=False, needs_layout_passes=False
        ),
    )
    def run(seq_hbm, tok_hbm, pos_hbm, out_hbm,
            idx_all, pos_all, rows0, rows1, x0, x1,
            gsem0, gsem1, osem0, osem1):
        rows_v = (rows0, rows1)
        xout_v = (x0, x1)
        gsem = (gsem0, gsem1)
        osem = (osem0, osem1)
        w = lax.axis_index("s") * NC + lax.axis_index("c")
        n_l = (L - 1 - w) // NW + 1
        nj = n_l * NCHUNK

        def stage(i, _):
            li = w + i * NW
            pltpu.sync_copy(seq_hbm.at[li], idx_all.at[i])
            pltpu.sync_copy(pos_hbm.at[li], pos_all.at[i])
            return 0

        lax.fori_loop(0, n_l, stage, 0)

        iota = lax.iota(jnp.int32, LANES)
        row16 = [iota + b0 for b0 in range(0, CHUNK, LANES)]

        def fire_gathers(j, b):
            li = j // NCHUNK
            c = j % NCHUNK
            for k in range(CHUNK // 128):
                pltpu.async_copy(
                    tok_hbm.at[idx_all.at[li, c * 2 + k]],
                    rows_v[b].at[pl.ds(k * 128, 128)],
                    gsem[b],
                )

        def drain(dummy_src, dst, sem):
            pltpu.make_async_copy(dummy_src, dst, sem).wait()

        fire_gathers(jnp.int32(0), 0)

        def half_body(jj, _):
            for b in (0, 1):
                j = jj * 2 + b
                li = j // NCHUNK
                c = j % NCHUNK
                l = w + li * NW

                @pl.when(j + 1 < nj)
                def _():
                    fire_gathers(j + 1, 1 - b)

                drain(tok_hbm.at[pl.ds(0, CHUNK)], rows_v[b], gsem[b])

                @pl.when(j >= 2)
                def _():
                    drain(out_hbm.at[0, :, pl.ds(0, CHUNK)], xout_v[b],
                          osem[b])

                def d_body(d, _):
                    pv = pos_all[li, d >> 3, pl.ds((d & 7) * LANES, LANES)]
                    colv = jnp.full((LANES,), d, jnp.int32)
                    gs = [
                        plsc.load_gather(rows_v[b], [row16[s], colv])
                        for s in range(CHUNK // LANES)
                    ]
                    for s in range(CHUNK // LANES):
                        xout_v[b][d, pl.ds(s * LANES, LANES)] = gs[s] + pv
                    return 0

                lax.fori_loop(0, D, d_body, 0)

                pltpu.async_copy(
                    xout_v[b],
                    out_hbm.at[l, :, pl.ds(c * CHUNK, CHUNK)],
                    osem[b],
                )
            return 0

        lax.fori_loop(0, nj // 2, half_body, 0)

        for b in (0, 1):
            drain(out_hbm.at[0, :, pl.ds(0, CHUNK)], xout_v[b], osem[b])

    return run


def kernel(seq, token_table, pos_table):
    b, l = seq.shape
    d = token_table.shape[1]
    seq3 = seq.T.reshape(l, b // 128, 128)
    pos16 = jnp.broadcast_to(
        pos_table[:, :, None], (l, d, LANES)
    ).reshape(l, d * LANES // 128, 128)
    out = _build()(seq3, token_table, pos16)
    return jnp.transpose(out, (2, 0, 1))
